# baseline (speedup 1.0000x reference)
import jax
import jax.numpy as jnp
from jax.experimental import pallas as pl

_N = 50000
_D = 256
_M = 512
_B = 200
_RS = 5
_CS = 1
_DC = _D // _CS
_G = _N // (_B * _RS)


def _mlp_segsum_kernel(*refs):
    x_refs = refs[:_RS * _CS]
    w1_ref, b1_ref, w2_ref, b2_ref, w3_ref, b3_ref = \
        refs[_RS * _CS:_RS * _CS + 6]
    ids_refs = refs[_RS * _CS + 6:_RS * _CS + 6 + _RS]
    out_ref = refs[-1]
    i = pl.program_id(0)

    w1 = w1_ref[...]
    partial = jnp.zeros((1, _M), dtype=jnp.float32)
    for k in range(_RS):
        u = b1_ref[...]
        for c in range(_CS):
            xc = x_refs[k * _CS + c][...].astype(jnp.bfloat16)
            u = u + jnp.dot(xc, w1[c * _DC:(c + 1) * _DC, :],
                            preferred_element_type=jnp.float32)
        u = u.astype(jnp.bfloat16)
        t = jnp.tanh(u)
        g = u + u * t
        u = (jnp.dot(g, w2_ref[...], preferred_element_type=jnp.float32)
             + b2_ref[...]).astype(jnp.bfloat16)
        t = jnp.tanh(u)
        h = (u + u * t).astype(jnp.float32)
        s = jnp.sum(h * w3_ref[...], axis=1, keepdims=True) + b3_ref[0, 0]

        ids = ids_refs[k][0, 0, :]
        seg = jax.lax.broadcasted_iota(jnp.int32, (_B, _M), 1)
        hit = ids[:, None] == seg
        partial = partial + jnp.sum(jnp.where(hit, s, 0.0), axis=0,
                                    keepdims=True)

    @pl.when(i == 0)
    def _():
        out_ref[...] = jnp.zeros_like(out_ref)

    out_ref[...] += partial


def _x_spec(k, c):
    return pl.BlockSpec((_B, _DC), lambda i, k=k, c=c: (_RS * i + k, c))


def _ids_spec(k):
    return pl.BlockSpec((1, 1, _B), lambda i, k=k: (_RS * i + k, 0, 0))


def kernel(energy, W1, b1, W2, b2, W3, b3, batch):
    ids3 = batch.astype(jnp.int32).reshape(_N // _B, 1, _B)
    w1h = (W1 * 0.5).astype(jnp.bfloat16)
    b1h = (b1 * 0.5).reshape(1, _D)
    w2h = (W2 * 0.5).astype(jnp.bfloat16)
    b2h = (b2 * 0.5).reshape(1, _D)
    out = pl.pallas_call(
        _mlp_segsum_kernel,
        grid=(_G,),
        in_specs=(
            [_x_spec(k, c) for k in range(_RS) for c in range(_CS)]
            + [
                pl.BlockSpec((_D, _D), lambda i: (0, 0)),
                pl.BlockSpec((1, _D), lambda i: (0, 0)),
                pl.BlockSpec((_D, _D), lambda i: (0, 0)),
                pl.BlockSpec((1, _D), lambda i: (0, 0)),
                pl.BlockSpec((1, _D), lambda i: (0, 0)),
                pl.BlockSpec((1, 1), lambda i: (0, 0)),
            ]
            + [_ids_spec(k) for k in range(_RS)]
        ),
        out_specs=pl.BlockSpec((1, _M), lambda i: (0, 0)),
        out_shape=jax.ShapeDtypeStruct((1, _M), jnp.float32),
    )(*([energy] * (_RS * _CS)
        + [w1h, b1h, w2h, b2h, W3.reshape(1, _D), b3.reshape(1, 1)]
        + [ids3] * _RS))
    return out[0]


# speedup vs baseline: 1.3683x; 1.3683x over previous
import jax
import jax.numpy as jnp
from jax.experimental import pallas as pl

_N = 50000
_D = 256
_M = 512
_B = 1000
_RS = 2
_CS = 1
_DC = _D // _CS
_G = _N // (_B * _RS)


def _mlp_segsum_kernel(*refs):
    x_refs = refs[:_RS * _CS]
    w1_ref, b1_ref, w2_ref, b2_ref, w3_ref, b3_ref = \
        refs[_RS * _CS:_RS * _CS + 6]
    ids_refs = refs[_RS * _CS + 6:_RS * _CS + 6 + _RS]
    out_ref = refs[-1]
    i = pl.program_id(0)

    w1 = w1_ref[...]
    partial = jnp.zeros((1, _M), dtype=jnp.float32)
    for k in range(_RS):
        u = b1_ref[...]
        for c in range(_CS):
            xc = x_refs[k * _CS + c][...].astype(jnp.bfloat16)
            u = u + jnp.dot(xc, w1[c * _DC:(c + 1) * _DC, :],
                            preferred_element_type=jnp.float32)
        u = u.astype(jnp.bfloat16)
        t = jnp.tanh(u)
        g = u + u * t
        u = (jnp.dot(g, w2_ref[...], preferred_element_type=jnp.float32)
             + b2_ref[...]).astype(jnp.bfloat16)
        t = jnp.tanh(u)
        h = (u + u * t).astype(jnp.float32)
        s = jnp.sum(h * w3_ref[...], axis=1, keepdims=True) + b3_ref[0, 0]

        ids = ids_refs[k][0, 0, :]
        seg = jax.lax.broadcasted_iota(jnp.int32, (_B, _M), 1)
        hit = ids[:, None] == seg
        partial = partial + jnp.sum(jnp.where(hit, s, 0.0), axis=0,
                                    keepdims=True)

    @pl.when(i == 0)
    def _():
        out_ref[...] = jnp.zeros_like(out_ref)

    out_ref[...] += partial


def _x_spec(k, c):
    return pl.BlockSpec((_B, _DC), lambda i, k=k, c=c: (_G * k + i, c))


def _ids_spec(k):
    return pl.BlockSpec((1, 1, _B), lambda i, k=k: (_G * k + i, 0, 0))


def kernel(energy, W1, b1, W2, b2, W3, b3, batch):
    ids3 = batch.astype(jnp.int32).reshape(_N // _B, 1, _B)
    w1h = (W1 * 0.5).astype(jnp.bfloat16)
    b1h = (b1 * 0.5).reshape(1, _D)
    w2h = (W2 * 0.5).astype(jnp.bfloat16)
    b2h = (b2 * 0.5).reshape(1, _D)
    out = pl.pallas_call(
        _mlp_segsum_kernel,
        grid=(_G,),
        in_specs=(
            [_x_spec(k, c) for k in range(_RS) for c in range(_CS)]
            + [
                pl.BlockSpec((_D, _D), lambda i: (0, 0)),
                pl.BlockSpec((1, _D), lambda i: (0, 0)),
                pl.BlockSpec((_D, _D), lambda i: (0, 0)),
                pl.BlockSpec((1, _D), lambda i: (0, 0)),
                pl.BlockSpec((1, _D), lambda i: (0, 0)),
                pl.BlockSpec((1, 1), lambda i: (0, 0)),
            ]
            + [_ids_spec(k) for k in range(_RS)]
        ),
        out_specs=pl.BlockSpec((1, _M), lambda i: (0, 0)),
        out_shape=jax.ShapeDtypeStruct((1, _M), jnp.float32),
    )(*([energy] * (_RS * _CS)
        + [w1h, b1h, w2h, b2h, W3.reshape(1, _D), b3.reshape(1, 1)]
        + [ids3] * _RS))
    return out[0]


# speedup vs baseline: 1.5008x; 1.0968x over previous
import jax
import jax.numpy as jnp
from jax.experimental import pallas as pl
from jax.experimental.pallas import tpu as pltpu

_N = 50000
_D = 256
_M = 512
_B = 1000
_RS = 2
_G = _N // (_B * _RS)


def _mlp_segsum_kernel(x0_ref, x1_ref, w1_ref, b1_ref, w2_ref, b2_ref,
                       w3_ref, b3_ref, ids0_ref, ids1_ref, out_ref,
                       w1s_ref, w2s_ref):
    i = pl.program_id(0)

    @pl.when(i == 0)
    def _():
        w1s_ref[...] = (w1_ref[...] * 0.5).astype(jnp.bfloat16)
        w2s_ref[...] = (w2_ref[...] * 0.5).astype(jnp.bfloat16)
        out_ref[...] = jnp.zeros_like(out_ref)

    b1h = b1_ref[...] * 0.5
    b2h = b2_ref[...] * 0.5
    w3r = w3_ref[...]
    b3 = b3_ref[0, 0]
    w1 = w1s_ref[...]
    w2 = w2s_ref[...]

    partial = jnp.zeros((1, _M), dtype=jnp.float32)
    for x_ref, ids_ref in ((x0_ref, ids0_ref), (x1_ref, ids1_ref)):
        x = x_ref[...].astype(jnp.bfloat16)
        u = (jnp.dot(x, w1, preferred_element_type=jnp.float32)
             + b1h).astype(jnp.bfloat16)
        t = jnp.tanh(u)
        g = u + u * t
        u = (jnp.dot(g, w2, preferred_element_type=jnp.float32)
             + b2h).astype(jnp.bfloat16)
        t = jnp.tanh(u)
        h = (u + u * t).astype(jnp.float32)
        s = jnp.sum(h * w3r, axis=1, keepdims=True) + b3

        ids = ids_ref[0, 0, :]
        seg = jax.lax.broadcasted_iota(jnp.int32, (_B, _M), 1)
        hit = ids[:, None] == seg
        partial = partial + jnp.sum(jnp.where(hit, s, 0.0), axis=0,
                                    keepdims=True)

    out_ref[...] += partial


def _x_spec(k):
    return pl.BlockSpec((_B, _D), lambda i, k=k: (_G * k + i, 0))


def _ids_spec(k):
    return pl.BlockSpec((1, 1, _B), lambda i, k=k: (_G * k + i, 0, 0))


def kernel(energy, W1, b1, W2, b2, W3, b3, batch):
    ids3 = batch.astype(jnp.int32).reshape(_N // _B, 1, _B)
    out = pl.pallas_call(
        _mlp_segsum_kernel,
        grid=(_G,),
        in_specs=[
            _x_spec(0),
            _x_spec(1),
            pl.BlockSpec((_D, _D), lambda i: (0, 0)),
            pl.BlockSpec((1, _D), lambda i: (0, 0)),
            pl.BlockSpec((_D, _D), lambda i: (0, 0)),
            pl.BlockSpec((1, _D), lambda i: (0, 0)),
            pl.BlockSpec((1, _D), lambda i: (0, 0)),
            pl.BlockSpec((1, 1), lambda i: (0, 0)),
            _ids_spec(0),
            _ids_spec(1),
        ],
        out_specs=pl.BlockSpec((1, _M), lambda i: (0, 0)),
        out_shape=jax.ShapeDtypeStruct((1, _M), jnp.float32),
        scratch_shapes=[
            pltpu.VMEM((_D, _D), jnp.bfloat16),
            pltpu.VMEM((_D, _D), jnp.bfloat16),
        ],
    )(energy, energy, W1, b1.reshape(1, _D), W2, b2.reshape(1, _D),
      W3.reshape(1, _D), b3.reshape(1, 1), ids3, ids3)
    return out[0]
